# baseline (speedup 1.0000x reference)
import functools

import jax
import jax.numpy as jnp
from jax import lax
from jax.experimental import pallas as pl
from jax.experimental.pallas import tpu as pltpu
from jax.experimental.pallas import tpu_sc as plsc

ROWS = 8192
D_IN = 768
D_LAT = 12288
K = 20
KC = 24
CW = 128
NCHUNKS = D_LAT // CW
BLK = 1024
CHUNK = 1024
NC = D_LAT // CHUNK
NEG = -3.4e38
BIGI = 2147480000

NWORKERS = 32
RPW = ROWS // NWORKERS
G = 4


def _enc_body(x_ref, we_ref, be_ref, lat_ref):
    lat_ref[...] = lax.dot_general(
        x_ref[...], we_ref[...], (((1,), (1,)), ((), ())),
        preferred_element_type=jnp.float32,
    ) + be_ref[...][None, :]


BLK2 = 256


def _sel_body(lat_ref, cid_ref, cmax_ref):
    M = jnp.max(lat_ref[...].reshape(BLK2, NCHUNKS, CW), axis=2)
    iota = lax.broadcasted_iota(jnp.int32, (BLK2, NCHUNKS), 1)
    for i in range(KC):
        m = jnp.max(M, axis=1, keepdims=True)
        cid = jnp.min(jnp.where(M == m, iota, NCHUNKS),
                      axis=1, keepdims=True)
        M = jnp.where(iota == cid, NEG, M)
        cid_ref[:, pl.ds(i, 1)] = cid
        cmax_ref[:, pl.ds(i, 1)] = m


def _encode(x, W_enc, b_enc):
    lat = pl.pallas_call(
        _enc_body,
        grid=(ROWS // BLK, NC),
        in_specs=[
            pl.BlockSpec((BLK, D_IN), lambda r, c: (r, 0)),
            pl.BlockSpec((CHUNK, D_IN), lambda r, c: (c, 0)),
            pl.BlockSpec((CHUNK,), lambda r, c: (c,)),
        ],
        out_specs=pl.BlockSpec((BLK, CHUNK), lambda r, c: (r, c)),
        out_shape=jax.ShapeDtypeStruct((ROWS, D_LAT), jnp.float32),
    )(x, W_enc, b_enc)
    cid, cmax = pl.pallas_call(
        _sel_body,
        grid=(ROWS // BLK2,),
        in_specs=[pl.BlockSpec((BLK2, D_LAT), lambda r: (r, 0))],
        out_specs=[
            pl.BlockSpec((BLK2, KC), lambda r: (r, 0)),
            pl.BlockSpec((BLK2, KC), lambda r: (r, 0)),
        ],
        out_shape=[
            jax.ShapeDtypeStruct((ROWS, KC), jnp.int32),
            jax.ShapeDtypeStruct((ROWS, KC), jnp.float32),
        ],
    )(lat)
    return lat, cid, cmax


def _splat(val):
    return jnp.full((16,), val)


def _sld(ref, i):
    return ref[pl.ds(i, 16)][0]


def _sc_body(latc, cid_hbm, cmax_hbm, wd_hbm, bd_hbm, out_hbm,
             cid_v, cmax_v, bd_v, gidx_v, chunks_v,
             selv_v, selc_v, wcol_v, wrows_v, outbuf_v, sem):
    wid = lax.axis_index("s") * 2 + lax.axis_index("c")
    base = wid * RPW
    pltpu.sync_copy(cid_hbm.at[pl.ds(base * KC, RPW * KC)],
                    cid_v.at[pl.ds(0, RPW * KC)])
    pltpu.sync_copy(cmax_hbm.at[pl.ds(base * KC, RPW * KC)],
                    cmax_v.at[pl.ds(0, RPW * KC)])
    pltpu.sync_copy(bd_hbm, bd_v)
    iota = lax.broadcasted_iota(jnp.int32, (16,), 0)
    lane0 = iota == 0

    def sst(ref, i, val):
        cur = ref[pl.ds(i, 16)]
        ref[pl.ds(i, 16)] = jnp.where(lane0, jnp.full((16,), val), cur)

    def group(grp, _):
        r0 = grp * G

        def gidx_row(g, _):
            r = r0 + g
            rowbase = _splat((base + r) * NCHUNKS)
            ca = cid_v[pl.ds(r * KC, 16)]
            cb = cid_v[pl.ds(r * KC + KC - 16, 16)]
            gidx_v[pl.ds(g * KC, 16)] = rowbase + ca
            gidx_v[pl.ds(g * KC + KC - 16, 16)] = rowbase + cb
            return 0
        lax.fori_loop(0, G, gidx_row, 0)
        pltpu.async_copy(latc.at[gidx_v], chunks_v, sem).wait()

        def select_row(g, _):
            r = r0 + g
            rm0 = cmax_v[pl.ds(r * KC, 16)]
            rm1 = cmax_v[pl.ds(r * KC + KC - 16, 16)]
            for i in range(K):
                m = jnp.maximum(jnp.max(rm0), jnp.max(rm1))
                msp = jnp.full((16,), m)
                u0 = jnp.min(jnp.where(rm0 == msp, iota, BIGI))
                u1 = jnp.min(jnp.where(rm1 == msp, iota + (KC - 16), BIGI))
                u = jnp.minimum(u0, u1)
                q = g * KC + u
                pos = jnp.int32(BIGI)
                for s in range(CW // 16):
                    v = chunks_v[q, pl.ds(s * 16, 16)]
                    pos = jnp.minimum(pos, jnp.min(
                        jnp.where(v == msp, s * 16 + iota, BIGI)))
                psp = jnp.full((16,), pos)
                nm = jnp.full((16,), NEG)
                for s in range(CW // 16):
                    v = chunks_v[q, pl.ds(s * 16, 16)]
                    kv = jnp.where((s * 16 + iota) == psp, NEG, v)
                    chunks_v[q, pl.ds(s * 16, 16)] = kv
                    nm = jnp.maximum(nm, kv)
                nmax = jnp.max(nm)
                usp = jnp.full((16,), u)
                rm0 = jnp.where(iota == usp, nmax, rm0)
                rm1 = jnp.where((iota + (KC - 16)) == usp, nmax, rm1)
                col = _sld(cid_v, r * KC + u) * CW + pos
                sst(selv_v, g * K + i, m)
                sst(selc_v, g * K + i, col)
            return 0
        lax.fori_loop(0, G, select_row, 0)

        for w in range(G * K // 16):
            wcol_v[pl.ds(w * 16, 16)] = selc_v[pl.ds(w * 16, 16)]
        pltpu.async_copy(wd_hbm.at[wcol_v], wrows_v, sem).wait()

        def decode_row(g, _):
            vals = [jnp.full((16,), _sld(selv_v, g * K + k))
                    for k in range(K)]
            for s in range(D_IN // 16):
                acc = bd_v[pl.ds(s * 16, 16)]
                for k in range(K):
                    acc = acc + vals[k] * wrows_v[g * K + k,
                                                  pl.ds(s * 16, 16)]
                outbuf_v[g, pl.ds(s * 16, 16)] = acc
            return 0
        lax.fori_loop(0, G, decode_row, 0)
        pltpu.sync_copy(outbuf_v, out_hbm.at[pl.ds(base + r0, G)])
        return 0

    lax.fori_loop(0, RPW // G, group, 0)


def _decode_sc(latc, cid, cmax, Wd, b_dec):
    mesh = plsc.VectorSubcoreMesh(core_axis_name="c", subcore_axis_name="s")
    f = functools.partial(
        pl.kernel,
        out_type=jax.ShapeDtypeStruct((ROWS, D_IN), jnp.float32),
        mesh=mesh,
        compiler_params=pltpu.CompilerParams(needs_layout_passes=False),
        scratch_types=[
            pltpu.VMEM((RPW * KC + 16,), jnp.int32),
            pltpu.VMEM((RPW * KC + 16,), jnp.float32),
            pltpu.VMEM((D_IN,), jnp.float32),
            pltpu.VMEM((G * KC,), jnp.int32),
            pltpu.VMEM((G * KC, CW), jnp.float32),
            pltpu.VMEM((G * K + 16,), jnp.float32),
            pltpu.VMEM((G * K + 16,), jnp.int32),
            pltpu.VMEM((G * K,), jnp.int32),
            pltpu.VMEM((G * K, D_IN), jnp.float32),
            pltpu.VMEM((G, D_IN), jnp.float32),
            pltpu.SemaphoreType.DMA,
        ],
    )
    return f(_sc_body)(latc, cid, cmax, Wd, b_dec)


def kernel(x, W_enc, b_enc, W_dec, b_dec):
    lat, cid, cmax = _encode(x, W_enc, b_enc)
    latc = lat.reshape(ROWS * NCHUNKS, CW)
    Wd = W_dec.T.reshape(D_LAT, D_IN)
    return _decode_sc(latc, cid.reshape(ROWS * KC), cmax.reshape(ROWS * KC),
                      Wd, b_dec)


# speedup vs baseline: 1.3081x; 1.3081x over previous
import functools

import jax
import jax.numpy as jnp
from jax import lax
from jax.experimental import pallas as pl
from jax.experimental.pallas import tpu as pltpu
from jax.experimental.pallas import tpu_sc as plsc

ROWS = 8192
D_IN = 768
D_LAT = 12288
K = 20
KC = 24
CW = 128
NCHUNKS = D_LAT // CW
BLK = 1024
CHUNK = 1024
NC = D_LAT // CHUNK
NEG = -3.4e38
BIGI = 2147480000

NWORKERS = 32
RPW = ROWS // NWORKERS
G = 2
NGRP = RPW // G
WCOPY = (0, 16, G * K - 16)


def _enc_body(x_ref, we_ref, be_ref, lat_ref):
    lat_ref[...] = lax.dot_general(
        x_ref[...], we_ref[...], (((1,), (1,)), ((), ())),
        preferred_element_type=jnp.float32,
    ) + be_ref[...][None, :]


BLK2 = 256


def _sel_body(lat_ref, cid_ref, cmax_ref):
    M = jnp.max(lat_ref[...].reshape(BLK2, NCHUNKS, CW), axis=2)
    iota = lax.broadcasted_iota(jnp.int32, (BLK2, NCHUNKS), 1)
    for i in range(KC):
        m = jnp.max(M, axis=1, keepdims=True)
        cid = jnp.min(jnp.where(M == m, iota, NCHUNKS),
                      axis=1, keepdims=True)
        M = jnp.where(iota == cid, NEG, M)
        cid_ref[:, pl.ds(i, 1)] = cid
        cmax_ref[:, pl.ds(i, 1)] = m


def _encode(x, W_enc, b_enc):
    lat = pl.pallas_call(
        _enc_body,
        grid=(ROWS // BLK, NC),
        in_specs=[
            pl.BlockSpec((BLK, D_IN), lambda r, c: (r, 0)),
            pl.BlockSpec((CHUNK, D_IN), lambda r, c: (c, 0)),
            pl.BlockSpec((CHUNK,), lambda r, c: (c,)),
        ],
        out_specs=pl.BlockSpec((BLK, CHUNK), lambda r, c: (r, c)),
        out_shape=jax.ShapeDtypeStruct((ROWS, D_LAT), jnp.float32),
    )(x, W_enc, b_enc)
    cid, cmax = pl.pallas_call(
        _sel_body,
        grid=(ROWS // BLK2,),
        in_specs=[pl.BlockSpec((BLK2, D_LAT), lambda r: (r, 0))],
        out_specs=[
            pl.BlockSpec((BLK2, KC), lambda r: (r, 0)),
            pl.BlockSpec((BLK2, KC), lambda r: (r, 0)),
        ],
        out_shape=[
            jax.ShapeDtypeStruct((ROWS, KC), jnp.int32),
            jax.ShapeDtypeStruct((ROWS, KC), jnp.float32),
        ],
    )(lat)
    return lat, cid, cmax


def _splat(val):
    return jnp.full((16,), val)


def _sld1(ref, i):
    return ref[pl.ds(i, 16)][0]


def _sc_body(latc, cid_hbm, cmax_hbm, wd_hbm, bd_hbm, out_hbm,
             cid_v, cmax_v, bd_v, gidx0_v, gidx1_v, chunks0_v, chunks1_v,
             selv0_v, selv1_v, selc0_v, selc1_v, wcol0_v, wcol1_v,
             wrows0_v, wrows1_v, outbuf0_v, outbuf1_v,
             semc0, semc1, semw0, semw1, semo0, semo1):
    wid = lax.axis_index("s") * 2 + lax.axis_index("c")
    base = wid * RPW
    pltpu.sync_copy(cid_hbm.at[pl.ds(base * KC, RPW * KC)],
                    cid_v.at[pl.ds(0, RPW * KC)])
    pltpu.sync_copy(cmax_hbm.at[pl.ds(base * KC, RPW * KC)],
                    cmax_v.at[pl.ds(0, RPW * KC)])
    pltpu.sync_copy(bd_hbm, bd_v)
    iota = lax.broadcasted_iota(jnp.int32, (16,), 0)
    lane0 = iota == 0
    semc = (semc0, semc1)
    semw = (semw0, semw1)
    semo = (semo0, semo1)
    gidx = (gidx0_v, gidx1_v)
    chunks = (chunks0_v, chunks1_v)
    selv = (selv0_v, selv1_v)
    selc = (selc0_v, selc1_v)
    wcol = (wcol0_v, wcol1_v)
    wrows = (wrows0_v, wrows1_v)
    outbuf = (outbuf0_v, outbuf1_v)

    def sst(ref, i, val):
        cur = ref[pl.ds(i, 16)]
        ref[pl.ds(i, 16)] = jnp.where(lane0, jnp.full((16,), val), cur)

    def build_gidx(b, g):
        def row(gr, _):
            r = g * G + gr
            rowbase = _splat((base + r) * NCHUNKS)
            ca = cid_v[pl.ds(r * KC, 16)]
            cb = cid_v[pl.ds(r * KC + KC - 16, 16)]
            gidx[b][pl.ds(gr * KC, 16)] = rowbase + ca
            gidx[b][pl.ds(gr * KC + KC - 16, 16)] = rowbase + cb
            return 0
        lax.fori_loop(0, G, row, 0)

    def fire_chunks(b):
        pltpu.async_copy(latc.at[gidx[b]], chunks[b], semc[b])

    def wait_chunks(b):
        pltpu.make_async_copy(latc.at[gidx[b]], chunks[b], semc[b]).wait()

    def select_group(b, g):
        def srow(gr, _):
            r = g * G + gr
            rm0i = cmax_v[pl.ds(r * KC, 16)]
            rm1i = cmax_v[pl.ds(r * KC + KC - 16, 16)]

            def extract(i, carry):
                rm0, rm1 = carry
                m = jnp.maximum(jnp.max(rm0), jnp.max(rm1))
                msp = jnp.full((16,), m)
                u0 = jnp.min(jnp.where(rm0 == msp, iota, BIGI))
                u1 = jnp.min(jnp.where(rm1 == msp, iota + (KC - 16), BIGI))
                u = jnp.minimum(u0, u1)
                q = gr * KC + u
                pos = jnp.int32(BIGI)
                for s in range(CW // 16):
                    v = chunks[b][q, pl.ds(s * 16, 16)]
                    pos = jnp.minimum(pos, jnp.min(
                        jnp.where(v == msp, s * 16 + iota, BIGI)))
                psp = jnp.full((16,), pos)
                nm = jnp.full((16,), NEG)
                for s in range(CW // 16):
                    v = chunks[b][q, pl.ds(s * 16, 16)]
                    kv = jnp.where((s * 16 + iota) == psp, NEG, v)
                    chunks[b][q, pl.ds(s * 16, 16)] = kv
                    nm = jnp.maximum(nm, kv)
                nmax = jnp.max(nm)
                usp = jnp.full((16,), u)
                rm0 = jnp.where(iota == usp, nmax, rm0)
                rm1 = jnp.where((iota + (KC - 16)) == usp, nmax, rm1)
                col = _sld1(cid_v, r * KC + u) * CW + pos
                sst(selv[b], gr * K + i, m)
                sst(selc[b], gr * K + i, col)
                return (rm0, rm1)
            lax.fori_loop(0, K, extract, (rm0i, rm1i))
            return 0
        lax.fori_loop(0, G, srow, 0)

    def fire_wd(b):
        for o in WCOPY:
            wcol[b][pl.ds(o, 16)] = selc[b][pl.ds(o, 16)]
        pltpu.async_copy(wd_hbm.at[wcol[b]], wrows[b], semw[b])

    def wait_wd(b):
        pltpu.make_async_copy(wd_hbm.at[wcol[b]], wrows[b], semw[b]).wait()

    def decode_group(b, g):
        def drow(gr, _):
            vals = [jnp.full((16,), _sld1(selv[b], gr * K + k))
                    for k in range(K)]
            for s in range(D_IN // 16):
                acc = bd_v[pl.ds(s * 16, 16)]
                for k in range(K):
                    acc = acc + vals[k] * wrows[b][gr * K + k,
                                                   pl.ds(s * 16, 16)]
                outbuf[b][gr, pl.ds(s * 16, 16)] = acc
            return 0
        lax.fori_loop(0, G, drow, 0)
        pltpu.async_copy(outbuf[b],
                         out_hbm.at[pl.ds(base + g * G, G)], semo[b])

    def drain_out(b, g):
        pltpu.make_async_copy(outbuf[b],
                              out_hbm.at[pl.ds(base + g * G, G)],
                              semo[b]).wait()

    for b in (0, 1):
        build_gidx(b, b)
        fire_chunks(b)

    def gg_body(gg, _):
        for b in (0, 1):
            g = gg * 2 + b
            bp = 1 - b
            wait_chunks(b)
            select_group(b, g)
            fire_wd(b)

            @pl.when(g + 2 < NGRP)
            def _prefetch():
                build_gidx(b, g + 2)
                fire_chunks(b)

            @pl.when(g >= 1)
            def _decode_prev():
                @pl.when(g >= 3)
                def _drain_prev():
                    drain_out(bp, g - 3)
                wait_wd(bp)
                decode_group(bp, g - 1)
        return 0
    lax.fori_loop(0, NGRP // 2, gg_body, 0)

    drain_out(1, NGRP - 3)
    wait_wd(1)
    decode_group(1, NGRP - 1)
    drain_out(0, NGRP - 2)
    drain_out(1, NGRP - 1)


def _decode_sc(latc, cid, cmax, Wd, b_dec):
    mesh = plsc.VectorSubcoreMesh(core_axis_name="c", subcore_axis_name="s")
    f = functools.partial(
        pl.kernel,
        out_type=jax.ShapeDtypeStruct((ROWS, D_IN), jnp.float32),
        mesh=mesh,
        compiler_params=pltpu.CompilerParams(needs_layout_passes=False),
        scratch_types=[
            pltpu.VMEM((RPW * KC + 16,), jnp.int32),
            pltpu.VMEM((RPW * KC + 16,), jnp.float32),
            pltpu.VMEM((D_IN,), jnp.float32),
            pltpu.VMEM((G * KC,), jnp.int32),
            pltpu.VMEM((G * KC,), jnp.int32),
            pltpu.VMEM((G * KC, CW), jnp.float32),
            pltpu.VMEM((G * KC, CW), jnp.float32),
            pltpu.VMEM((G * K + 16,), jnp.float32),
            pltpu.VMEM((G * K + 16,), jnp.float32),
            pltpu.VMEM((G * K + 16,), jnp.int32),
            pltpu.VMEM((G * K + 16,), jnp.int32),
            pltpu.VMEM((G * K,), jnp.int32),
            pltpu.VMEM((G * K,), jnp.int32),
            pltpu.VMEM((G * K, D_IN), jnp.float32),
            pltpu.VMEM((G * K, D_IN), jnp.float32),
            pltpu.VMEM((G, D_IN), jnp.float32),
            pltpu.VMEM((G, D_IN), jnp.float32),
            pltpu.SemaphoreType.DMA,
            pltpu.SemaphoreType.DMA,
            pltpu.SemaphoreType.DMA,
            pltpu.SemaphoreType.DMA,
            pltpu.SemaphoreType.DMA,
            pltpu.SemaphoreType.DMA,
        ],
    )
    return f(_sc_body)(latc, cid, cmax, Wd, b_dec)


def kernel(x, W_enc, b_enc, W_dec, b_dec):
    lat, cid, cmax = _encode(x, W_enc, b_enc)
    latc = lat.reshape(ROWS * NCHUNKS, CW)
    Wd = W_dec.T.reshape(D_LAT, D_IN)
    return _decode_sc(latc, cid.reshape(ROWS * KC), cmax.reshape(ROWS * KC),
                      Wd, b_dec)


# speedup vs baseline: 1.3313x; 1.0177x over previous
import functools

import jax
import jax.numpy as jnp
from jax import lax
from jax.experimental import pallas as pl
from jax.experimental.pallas import tpu as pltpu
from jax.experimental.pallas import tpu_sc as plsc

ROWS = 8192
D_IN = 768
D_LAT = 12288
K = 20
KC = 24
CW = 128
NCHUNKS = D_LAT // CW
BLK = 1024
CHUNK = 1024
NC = D_LAT // CHUNK
NEG = -3.4e38
BIGI = 2147480000

NWORKERS = 32
RPW = ROWS // NWORKERS
G = 2
NGRP = RPW // G
WCOPY = (0, 16, G * K - 16)


def _enc_body(x_ref, we_ref, be_ref, lat_ref):
    lat_ref[...] = lax.dot_general(
        x_ref[...], we_ref[...], (((1,), (1,)), ((), ())),
        preferred_element_type=jnp.float32,
    ) + be_ref[...][None, :]


BLK2 = 256


def _sel_body(lat_ref, cid_ref, cmax_ref):
    M = jnp.max(lat_ref[...].reshape(BLK2, NCHUNKS, CW), axis=2)
    iota = lax.broadcasted_iota(jnp.int32, (BLK2, NCHUNKS), 1)
    for i in range(KC):
        m = jnp.max(M, axis=1, keepdims=True)
        cid = jnp.min(jnp.where(M == m, iota, NCHUNKS),
                      axis=1, keepdims=True)
        M = jnp.where(iota == cid, NEG, M)
        cid_ref[:, pl.ds(i, 1)] = cid
        cmax_ref[:, pl.ds(i, 1)] = m


def _encode(x, W_enc, b_enc):
    lat = pl.pallas_call(
        _enc_body,
        grid=(ROWS // BLK, NC),
        in_specs=[
            pl.BlockSpec((BLK, D_IN), lambda r, c: (r, 0)),
            pl.BlockSpec((CHUNK, D_IN), lambda r, c: (c, 0)),
            pl.BlockSpec((CHUNK,), lambda r, c: (c,)),
        ],
        out_specs=pl.BlockSpec((BLK, CHUNK), lambda r, c: (r, c)),
        out_shape=jax.ShapeDtypeStruct((ROWS, D_LAT), jnp.float32),
    )(x, W_enc, b_enc)
    cid, cmax = pl.pallas_call(
        _sel_body,
        grid=(ROWS // BLK2,),
        in_specs=[pl.BlockSpec((BLK2, D_LAT), lambda r: (r, 0))],
        out_specs=[
            pl.BlockSpec((BLK2, KC), lambda r: (r, 0)),
            pl.BlockSpec((BLK2, KC), lambda r: (r, 0)),
        ],
        out_shape=[
            jax.ShapeDtypeStruct((ROWS, KC), jnp.int32),
            jax.ShapeDtypeStruct((ROWS, KC), jnp.float32),
        ],
    )(lat)
    return lat, cid, cmax


def _splat(val):
    return jnp.full((16,), val)


def _sld1(ref, i):
    return ref[pl.ds(i, 16)][0]


def _sc_body(latc, cid_hbm, cmax_hbm, wd_hbm, bd_hbm, out_hbm,
             cid_v, cmax_v, bd_v, gidx0_v, gidx1_v, chunks0_v, chunks1_v,
             selv0_v, selv1_v, selc0_v, selc1_v, wcol0_v, wcol1_v,
             wrows0_v, wrows1_v, outbuf0_v, outbuf1_v,
             semc0, semc1, semw0, semw1, semo0, semo1):
    wid = lax.axis_index("s") * 2 + lax.axis_index("c")
    base = wid * RPW
    pltpu.sync_copy(cid_hbm.at[pl.ds(base * KC, RPW * KC)],
                    cid_v.at[pl.ds(0, RPW * KC)])
    pltpu.sync_copy(cmax_hbm.at[pl.ds(base * KC, RPW * KC)],
                    cmax_v.at[pl.ds(0, RPW * KC)])
    pltpu.sync_copy(bd_hbm, bd_v)
    iota = lax.broadcasted_iota(jnp.int32, (16,), 0)
    lane0 = iota == 0
    semc = (semc0, semc1)
    semw = (semw0, semw1)
    semo = (semo0, semo1)
    gidx = (gidx0_v, gidx1_v)
    chunks = (chunks0_v, chunks1_v)
    selv = (selv0_v, selv1_v)
    selc = (selc0_v, selc1_v)
    wcol = (wcol0_v, wcol1_v)
    wrows = (wrows0_v, wrows1_v)
    outbuf = (outbuf0_v, outbuf1_v)

    def sst(ref, i, val):
        cur = ref[pl.ds(i, 16)]
        ref[pl.ds(i, 16)] = jnp.where(lane0, jnp.full((16,), val), cur)

    def build_gidx(b, g):
        def row(gr, _):
            r = g * G + gr
            rowbase = _splat((base + r) * NCHUNKS)
            ca = cid_v[pl.ds(r * KC, 16)]
            cb = cid_v[pl.ds(r * KC + KC - 16, 16)]
            gidx[b][pl.ds(gr * KC, 16)] = rowbase + ca
            gidx[b][pl.ds(gr * KC + KC - 16, 16)] = rowbase + cb
            return 0
        lax.fori_loop(0, G, row, 0)

    def fire_chunks(b):
        pltpu.async_copy(latc.at[gidx[b]], chunks[b], semc[b])

    def wait_chunks(b):
        pltpu.make_async_copy(latc.at[gidx[b]], chunks[b], semc[b]).wait()

    def select_group(b, g):
        def row_step(gr, i, rm0, rm1):
            r = g * G + gr
            m = jnp.maximum(jnp.max(rm0), jnp.max(rm1))
            msp = jnp.full((16,), m)
            u0 = jnp.min(jnp.where(rm0 == msp, iota, BIGI))
            u1 = jnp.min(jnp.where(rm1 == msp, iota + (KC - 16), BIGI))
            u = jnp.minimum(u0, u1)
            q = gr * KC + u
            pos = jnp.int32(BIGI)
            for s in range(CW // 16):
                v = chunks[b][q, pl.ds(s * 16, 16)]
                pos = jnp.minimum(pos, jnp.min(
                    jnp.where(v == msp, s * 16 + iota, BIGI)))
            psp = jnp.full((16,), pos)
            nm = jnp.full((16,), NEG)
            for s in range(CW // 16):
                v = chunks[b][q, pl.ds(s * 16, 16)]
                kv = jnp.where((s * 16 + iota) == psp, NEG, v)
                chunks[b][q, pl.ds(s * 16, 16)] = kv
                nm = jnp.maximum(nm, kv)
            nmax = jnp.max(nm)
            usp = jnp.full((16,), u)
            rm0 = jnp.where(iota == usp, nmax, rm0)
            rm1 = jnp.where((iota + (KC - 16)) == usp, nmax, rm1)
            col = _sld1(cid_v, r * KC + u) * CW + pos
            sst(selv[b], gr * K + i, m)
            sst(selc[b], gr * K + i, col)
            return rm0, rm1

        rms = []
        for gr in range(G):
            r = g * G + gr
            rms.append(cmax_v[pl.ds(r * KC, 16)])
            rms.append(cmax_v[pl.ds(r * KC + KC - 16, 16)])

        def extract(i, carry):
            out = []
            for gr in range(G):
                a, c = row_step(gr, i, carry[2 * gr], carry[2 * gr + 1])
                out += [a, c]
            return tuple(out)
        lax.fori_loop(0, K, extract, tuple(rms))

    def fire_wd(b):
        for o in WCOPY:
            wcol[b][pl.ds(o, 16)] = selc[b][pl.ds(o, 16)]
        pltpu.async_copy(wd_hbm.at[wcol[b]], wrows[b], semw[b])

    def wait_wd(b):
        pltpu.make_async_copy(wd_hbm.at[wcol[b]], wrows[b], semw[b]).wait()

    def decode_group(b, g):
        def drow(gr, _):
            vals = [jnp.full((16,), _sld1(selv[b], gr * K + k))
                    for k in range(K)]
            for s in range(D_IN // 16):
                acc = bd_v[pl.ds(s * 16, 16)]
                for k in range(K):
                    acc = acc + vals[k] * wrows[b][gr * K + k,
                                                   pl.ds(s * 16, 16)]
                outbuf[b][gr, pl.ds(s * 16, 16)] = acc
            return 0
        lax.fori_loop(0, G, drow, 0)
        pltpu.async_copy(outbuf[b],
                         out_hbm.at[pl.ds(base + g * G, G)], semo[b])

    def drain_out(b, g):
        pltpu.make_async_copy(outbuf[b],
                              out_hbm.at[pl.ds(base + g * G, G)],
                              semo[b]).wait()

    for b in (0, 1):
        build_gidx(b, b)
        fire_chunks(b)

    def gg_body(gg, _):
        for b in (0, 1):
            g = gg * 2 + b
            bp = 1 - b
            wait_chunks(b)
            select_group(b, g)
            fire_wd(b)

            @pl.when(g + 2 < NGRP)
            def _prefetch():
                build_gidx(b, g + 2)
                fire_chunks(b)

            @pl.when(g >= 1)
            def _decode_prev():
                @pl.when(g >= 3)
                def _drain_prev():
                    drain_out(bp, g - 3)
                wait_wd(bp)
                decode_group(bp, g - 1)
        return 0
    lax.fori_loop(0, NGRP // 2, gg_body, 0)

    drain_out(1, NGRP - 3)
    wait_wd(1)
    decode_group(1, NGRP - 1)
    drain_out(0, NGRP - 2)
    drain_out(1, NGRP - 1)


def _decode_sc(latc, cid, cmax, Wd, b_dec):
    mesh = plsc.VectorSubcoreMesh(core_axis_name="c", subcore_axis_name="s")
    f = functools.partial(
        pl.kernel,
        out_type=jax.ShapeDtypeStruct((ROWS, D_IN), jnp.float32),
        mesh=mesh,
        compiler_params=pltpu.CompilerParams(needs_layout_passes=False),
        scratch_types=[
            pltpu.VMEM((RPW * KC + 16,), jnp.int32),
            pltpu.VMEM((RPW * KC + 16,), jnp.float32),
            pltpu.VMEM((D_IN,), jnp.float32),
            pltpu.VMEM((G * KC,), jnp.int32),
            pltpu.VMEM((G * KC,), jnp.int32),
            pltpu.VMEM((G * KC, CW), jnp.float32),
            pltpu.VMEM((G * KC, CW), jnp.float32),
            pltpu.VMEM((G * K + 16,), jnp.float32),
            pltpu.VMEM((G * K + 16,), jnp.float32),
            pltpu.VMEM((G * K + 16,), jnp.int32),
            pltpu.VMEM((G * K + 16,), jnp.int32),
            pltpu.VMEM((G * K,), jnp.int32),
            pltpu.VMEM((G * K,), jnp.int32),
            pltpu.VMEM((G * K, D_IN), jnp.float32),
            pltpu.VMEM((G * K, D_IN), jnp.float32),
            pltpu.VMEM((G, D_IN), jnp.float32),
            pltpu.VMEM((G, D_IN), jnp.float32),
            pltpu.SemaphoreType.DMA,
            pltpu.SemaphoreType.DMA,
            pltpu.SemaphoreType.DMA,
            pltpu.SemaphoreType.DMA,
            pltpu.SemaphoreType.DMA,
            pltpu.SemaphoreType.DMA,
        ],
    )
    return f(_sc_body)(latc, cid, cmax, Wd, b_dec)


def kernel(x, W_enc, b_enc, W_dec, b_dec):
    lat, cid, cmax = _encode(x, W_enc, b_enc)
    latc = lat.reshape(ROWS * NCHUNKS, CW)
    Wd = W_dec.T.reshape(D_LAT, D_IN)
    return _decode_sc(latc, cid.reshape(ROWS * KC), cmax.reshape(ROWS * KC),
                      Wd, b_dec)


# speedup vs baseline: 1.6296x; 1.2241x over previous
import functools

import jax
import jax.numpy as jnp
from jax import lax
from jax.experimental import pallas as pl
from jax.experimental.pallas import tpu as pltpu
from jax.experimental.pallas import tpu_sc as plsc

ROWS = 8192
D_IN = 768
D_LAT = 12288
K = 20
KC = 24
CW = 128
NCHUNKS = D_LAT // CW
BLK = 1024
CHUNK = 1024
NC = D_LAT // CHUNK
NEG = -3.4e38
BIGI = 2147480000

NWORKERS = 32
RPW = ROWS // NWORKERS
G = 2
NGRP = RPW // G
WCOPY = (0, 16, G * K - 16)


def _enc_body(x_ref, we_ref, be_ref, lat_ref):
    lat_ref[...] = lax.dot_general(
        x_ref[...], we_ref[...], (((1,), (1,)), ((), ())),
        preferred_element_type=jnp.float32,
    ) + be_ref[...][None, :]


BLK2 = 256


def _sel_body(lat_ref, cid_ref, cmax_ref):
    M = jnp.max(lat_ref[...].reshape(BLK2, NCHUNKS, CW), axis=2)
    iota = lax.broadcasted_iota(jnp.int32, (BLK2, NCHUNKS), 1)
    for i in range(KC):
        m = jnp.max(M, axis=1, keepdims=True)
        cid = jnp.min(jnp.where(M == m, iota, NCHUNKS),
                      axis=1, keepdims=True)
        M = jnp.where(iota == cid, NEG, M)
        cid_ref[:, pl.ds(i, 1)] = cid
        cmax_ref[:, pl.ds(i, 1)] = m


def _encode(x, W_enc, b_enc, nrows):
    lat = pl.pallas_call(
        _enc_body,
        grid=(nrows // BLK, NC),
        in_specs=[
            pl.BlockSpec((BLK, D_IN), lambda r, c: (r, 0)),
            pl.BlockSpec((CHUNK, D_IN), lambda r, c: (c, 0)),
            pl.BlockSpec((CHUNK,), lambda r, c: (c,)),
        ],
        out_specs=pl.BlockSpec((BLK, CHUNK), lambda r, c: (r, c)),
        out_shape=jax.ShapeDtypeStruct((nrows, D_LAT), jnp.float32),
    )(x, W_enc, b_enc)
    cid, cmax = pl.pallas_call(
        _sel_body,
        grid=(nrows // BLK2,),
        in_specs=[pl.BlockSpec((BLK2, D_LAT), lambda r: (r, 0))],
        out_specs=[
            pl.BlockSpec((BLK2, KC), lambda r: (r, 0)),
            pl.BlockSpec((BLK2, KC), lambda r: (r, 0)),
        ],
        out_shape=[
            jax.ShapeDtypeStruct((nrows, KC), jnp.int32),
            jax.ShapeDtypeStruct((nrows, KC), jnp.float32),
        ],
    )(lat)
    return lat, cid, cmax


def _splat(val):
    return jnp.full((16,), val)


def _sld1(ref, i):
    return ref[pl.ds(i, 16)][0]


def _make_sc_body(rpw, ngrp):
  def _sc_body(latc, cid_hbm, cmax_hbm, wd_hbm, bd_hbm, out_hbm,
             cid_v, cmax_v, bd_v, gidx0_v, gidx1_v, chunks0_v, chunks1_v,
             selv0_v, selv1_v, selc0_v, selc1_v, wcol0_v, wcol1_v,
             wrows0_v, wrows1_v, outbuf0_v, outbuf1_v,
             semc0, semc1, semw0, semw1, semo0, semo1):
    wid = lax.axis_index("s") * 2 + lax.axis_index("c")
    base = wid * rpw
    pltpu.sync_copy(cid_hbm.at[pl.ds(base * KC, rpw * KC)],
                    cid_v.at[pl.ds(0, rpw * KC)])
    pltpu.sync_copy(cmax_hbm.at[pl.ds(base * KC, rpw * KC)],
                    cmax_v.at[pl.ds(0, rpw * KC)])
    pltpu.sync_copy(bd_hbm, bd_v)
    iota = lax.broadcasted_iota(jnp.int32, (16,), 0)
    lane0 = iota == 0
    semc = (semc0, semc1)
    semw = (semw0, semw1)
    semo = (semo0, semo1)
    gidx = (gidx0_v, gidx1_v)
    chunks = (chunks0_v, chunks1_v)
    selv = (selv0_v, selv1_v)
    selc = (selc0_v, selc1_v)
    wcol = (wcol0_v, wcol1_v)
    wrows = (wrows0_v, wrows1_v)
    outbuf = (outbuf0_v, outbuf1_v)

    def sst(ref, i, val):
        cur = ref[pl.ds(i, 16)]
        ref[pl.ds(i, 16)] = jnp.where(lane0, jnp.full((16,), val), cur)

    def build_gidx(b, g):
        def row(gr, _):
            r = g * G + gr
            rowbase = _splat((base + r) * NCHUNKS)
            ca = cid_v[pl.ds(r * KC, 16)]
            cb = cid_v[pl.ds(r * KC + KC - 16, 16)]
            gidx[b][pl.ds(gr * KC, 16)] = rowbase + ca
            gidx[b][pl.ds(gr * KC + KC - 16, 16)] = rowbase + cb
            return 0
        lax.fori_loop(0, G, row, 0)

    def fire_chunks(b):
        pltpu.async_copy(latc.at[gidx[b]], chunks[b], semc[b])

    def wait_chunks(b):
        pltpu.make_async_copy(latc.at[gidx[b]], chunks[b], semc[b]).wait()

    def select_group(b, g):
        def row_step(gr, i, rm0, rm1):
            r = g * G + gr
            m = jnp.maximum(jnp.max(rm0), jnp.max(rm1))
            msp = jnp.full((16,), m)
            u0 = jnp.min(jnp.where(rm0 == msp, iota, BIGI))
            u1 = jnp.min(jnp.where(rm1 == msp, iota + (KC - 16), BIGI))
            u = jnp.minimum(u0, u1)
            q = gr * KC + u
            pos = jnp.int32(BIGI)
            for s in range(CW // 16):
                v = chunks[b][q, pl.ds(s * 16, 16)]
                pos = jnp.minimum(pos, jnp.min(
                    jnp.where(v == msp, s * 16 + iota, BIGI)))
            psp = jnp.full((16,), pos)
            nm = jnp.full((16,), NEG)
            for s in range(CW // 16):
                v = chunks[b][q, pl.ds(s * 16, 16)]
                kv = jnp.where((s * 16 + iota) == psp, NEG, v)
                chunks[b][q, pl.ds(s * 16, 16)] = kv
                nm = jnp.maximum(nm, kv)
            nmax = jnp.max(nm)
            usp = jnp.full((16,), u)
            rm0 = jnp.where(iota == usp, nmax, rm0)
            rm1 = jnp.where((iota + (KC - 16)) == usp, nmax, rm1)
            col = _sld1(cid_v, r * KC + u) * CW + pos
            sst(selv[b], gr * K + i, m)
            sst(selc[b], gr * K + i, col)
            return rm0, rm1

        rms = []
        for gr in range(G):
            r = g * G + gr
            rms.append(cmax_v[pl.ds(r * KC, 16)])
            rms.append(cmax_v[pl.ds(r * KC + KC - 16, 16)])

        def extract(i, carry):
            out = []
            for gr in range(G):
                a, c = row_step(gr, i, carry[2 * gr], carry[2 * gr + 1])
                out += [a, c]
            return tuple(out)
        lax.fori_loop(0, K, extract, tuple(rms))

    def fire_wd(b):
        for o in WCOPY:
            wcol[b][pl.ds(o, 16)] = selc[b][pl.ds(o, 16)]
        pltpu.async_copy(wd_hbm.at[wcol[b]], wrows[b], semw[b])

    def wait_wd(b):
        pltpu.make_async_copy(wd_hbm.at[wcol[b]], wrows[b], semw[b]).wait()

    def decode_group(b, g):
        def drow(gr, _):
            vals = [jnp.full((16,), _sld1(selv[b], gr * K + k))
                    for k in range(K)]
            for s in range(D_IN // 16):
                acc = bd_v[pl.ds(s * 16, 16)]
                for k in range(K):
                    acc = acc + vals[k] * wrows[b][gr * K + k,
                                                   pl.ds(s * 16, 16)]
                outbuf[b][gr, pl.ds(s * 16, 16)] = acc
            return 0
        lax.fori_loop(0, G, drow, 0)
        pltpu.async_copy(outbuf[b],
                         out_hbm.at[pl.ds(base + g * G, G)], semo[b])

    def drain_out(b, g):
        pltpu.make_async_copy(outbuf[b],
                              out_hbm.at[pl.ds(base + g * G, G)],
                              semo[b]).wait()

    for b in (0, 1):
        build_gidx(b, b)
        fire_chunks(b)

    def gg_body(gg, _):
        for b in (0, 1):
            g = gg * 2 + b
            bp = 1 - b
            wait_chunks(b)
            select_group(b, g)
            fire_wd(b)

            @pl.when(g + 2 < ngrp)
            def _prefetch():
                build_gidx(b, g + 2)
                fire_chunks(b)

            @pl.when(g >= 1)
            def _decode_prev():
                @pl.when(g >= 3)
                def _drain_prev():
                    drain_out(bp, g - 3)
                wait_wd(bp)
                decode_group(bp, g - 1)
        return 0
    lax.fori_loop(0, ngrp // 2, gg_body, 0)

    drain_out(1, ngrp - 3)
    wait_wd(1)
    decode_group(1, ngrp - 1)
    drain_out(0, ngrp - 2)
    drain_out(1, ngrp - 1)
  return _sc_body


def _decode_sc(latc, cid, cmax, Wd, b_dec, nrows):
    rpw = nrows // NWORKERS
    ngrp = rpw // G
    mesh = plsc.VectorSubcoreMesh(core_axis_name="c", subcore_axis_name="s")
    f = functools.partial(
        pl.kernel,
        out_type=jax.ShapeDtypeStruct((nrows, D_IN), jnp.float32),
        mesh=mesh,
        compiler_params=pltpu.CompilerParams(needs_layout_passes=False),
        scratch_types=[
            pltpu.VMEM((rpw * KC + 16,), jnp.int32),
            pltpu.VMEM((rpw * KC + 16,), jnp.float32),
            pltpu.VMEM((D_IN,), jnp.float32),
            pltpu.VMEM((G * KC,), jnp.int32),
            pltpu.VMEM((G * KC,), jnp.int32),
            pltpu.VMEM((G * KC, CW), jnp.float32),
            pltpu.VMEM((G * KC, CW), jnp.float32),
            pltpu.VMEM((G * K + 16,), jnp.float32),
            pltpu.VMEM((G * K + 16,), jnp.float32),
            pltpu.VMEM((G * K + 16,), jnp.int32),
            pltpu.VMEM((G * K + 16,), jnp.int32),
            pltpu.VMEM((G * K,), jnp.int32),
            pltpu.VMEM((G * K,), jnp.int32),
            pltpu.VMEM((G * K, D_IN), jnp.float32),
            pltpu.VMEM((G * K, D_IN), jnp.float32),
            pltpu.VMEM((G, D_IN), jnp.float32),
            pltpu.VMEM((G, D_IN), jnp.float32),
            pltpu.SemaphoreType.DMA,
            pltpu.SemaphoreType.DMA,
            pltpu.SemaphoreType.DMA,
            pltpu.SemaphoreType.DMA,
            pltpu.SemaphoreType.DMA,
            pltpu.SemaphoreType.DMA,
        ],
    )
    return f(_make_sc_body(rpw, ngrp))(latc, cid, cmax, Wd, b_dec)


def kernel(x, W_enc, b_enc, W_dec, b_dec):
    Wd = W_dec.T.reshape(D_LAT, D_IN)
    h = ROWS // 2
    outs = []
    for xh in (x[:h], x[h:]):
        lat, cid, cmax = _encode(xh, W_enc, b_enc, h)
        latc = lat.reshape(h * NCHUNKS, CW)
        outs.append(_decode_sc(latc, cid.reshape(h * KC),
                               cmax.reshape(h * KC), Wd, b_dec, h))
    return jnp.concatenate(outs, axis=0)


# speedup vs baseline: 1.8188x; 1.1161x over previous
import functools

import jax
import jax.numpy as jnp
from jax import lax
from jax.experimental import pallas as pl
from jax.experimental.pallas import tpu as pltpu
from jax.experimental.pallas import tpu_sc as plsc

ROWS = 8192
D_IN = 768
D_LAT = 12288
K = 20
KC = 24
CW = 128
NCHUNKS = D_LAT // CW
BLK = 1024
CHUNK = 1024
NC = D_LAT // CHUNK
NEG = -3.4e38
BIGI = 2147480000

NWORKERS = 32
RPW = ROWS // NWORKERS
G = 2
NGRP = RPW // G
WCOPY = (0, 16, G * K - 16)


def _enc_body(x_ref, we_ref, be_ref, lat_ref):
    lat_ref[...] = lax.dot_general(
        x_ref[...], we_ref[...], (((1,), (1,)), ((), ())),
        preferred_element_type=jnp.float32,
    ) + be_ref[...][None, :]


BLK2 = 256


def _sel_body(lat_ref, cid_ref, cmax_ref):
    M = jnp.max(lat_ref[...].reshape(BLK2, NCHUNKS, CW), axis=2)
    iota = lax.broadcasted_iota(jnp.int32, (BLK2, NCHUNKS), 1)
    for i in range(KC):
        m = jnp.max(M, axis=1, keepdims=True)
        cid = jnp.min(jnp.where(M == m, iota, NCHUNKS),
                      axis=1, keepdims=True)
        M = jnp.where(iota == cid, NEG, M)
        cid_ref[:, pl.ds(i, 1)] = cid
        cmax_ref[:, pl.ds(i, 1)] = m


def _encode(x, W_enc, b_enc, nrows):
    lat = pl.pallas_call(
        _enc_body,
        grid=(nrows // BLK, NC),
        in_specs=[
            pl.BlockSpec((BLK, D_IN), lambda r, c: (r, 0)),
            pl.BlockSpec((CHUNK, D_IN), lambda r, c: (c, 0)),
            pl.BlockSpec((CHUNK,), lambda r, c: (c,)),
        ],
        out_specs=pl.BlockSpec((BLK, CHUNK), lambda r, c: (r, c)),
        out_shape=jax.ShapeDtypeStruct((nrows, D_LAT), jnp.float32),
    )(x, W_enc, b_enc)
    cid, cmax = pl.pallas_call(
        _sel_body,
        grid=(nrows // BLK2,),
        in_specs=[pl.BlockSpec((BLK2, D_LAT), lambda r: (r, 0))],
        out_specs=[
            pl.BlockSpec((BLK2, KC), lambda r: (r, 0)),
            pl.BlockSpec((BLK2, KC), lambda r: (r, 0)),
        ],
        out_shape=[
            jax.ShapeDtypeStruct((nrows, KC), jnp.int32),
            jax.ShapeDtypeStruct((nrows, KC), jnp.float32),
        ],
    )(lat)
    return lat, cid, cmax


def _splat(val):
    return jnp.full((16,), val)


def _sld1(ref, i):
    return ref[pl.ds(i, 16)][0]


def _make_sc_body(rpw, ngrp):
  def _sc_body(latc, cid_hbm, cmax_hbm, wd_hbm, bd_hbm, out_hbm,
             cid_v, cmax_v, bd_v, gidx0_v, gidx1_v, chunks0_v, chunks1_v,
             selv0_v, selv1_v, selc0_v, selc1_v, wcol0_v, wcol1_v,
             wrows0_v, wrows1_v, outbuf0_v, outbuf1_v,
             semc0, semc1, semw0, semw1, semo0, semo1):
    wid = lax.axis_index("s") * 2 + lax.axis_index("c")
    base = wid * rpw
    pltpu.sync_copy(cid_hbm.at[pl.ds(base * KC, rpw * KC)],
                    cid_v.at[pl.ds(0, rpw * KC)])
    pltpu.sync_copy(cmax_hbm.at[pl.ds(base * KC, rpw * KC)],
                    cmax_v.at[pl.ds(0, rpw * KC)])
    pltpu.sync_copy(bd_hbm, bd_v)
    iota = lax.broadcasted_iota(jnp.int32, (16,), 0)
    lane0 = iota == 0
    semc = (semc0, semc1)
    semw = (semw0, semw1)
    semo = (semo0, semo1)
    gidx = (gidx0_v, gidx1_v)
    chunks = (chunks0_v, chunks1_v)
    selv = (selv0_v, selv1_v)
    selc = (selc0_v, selc1_v)
    wcol = (wcol0_v, wcol1_v)
    wrows = (wrows0_v, wrows1_v)
    outbuf = (outbuf0_v, outbuf1_v)

    def sst(ref, i, val):
        cur = ref[pl.ds(i, 16)]
        ref[pl.ds(i, 16)] = jnp.where(lane0, jnp.full((16,), val), cur)

    def build_gidx(b, g):
        def row(gr, _):
            r = g * G + gr
            rowbase = _splat((base + r) * NCHUNKS)
            ca = cid_v[pl.ds(r * KC, 16)]
            cb = cid_v[pl.ds(r * KC + KC - 16, 16)]
            gidx[b][pl.ds(gr * KC, 16)] = rowbase + ca
            gidx[b][pl.ds(gr * KC + KC - 16, 16)] = rowbase + cb
            return 0
        lax.fori_loop(0, G, row, 0)

    def fire_chunks(b):
        pltpu.async_copy(latc.at[gidx[b]], chunks[b], semc[b])

    def wait_chunks(b):
        pltpu.make_async_copy(latc.at[gidx[b]], chunks[b], semc[b]).wait()

    def select_group(b, g):
        def row_step(gr, i, rm0, rm1):
            r = g * G + gr
            m = jnp.maximum(jnp.max(rm0), jnp.max(rm1))
            msp = jnp.full((16,), m)
            u0 = jnp.min(jnp.where(rm0 == msp, iota, BIGI))
            u1 = jnp.min(jnp.where(rm1 == msp, iota + (KC - 16), BIGI))
            u = jnp.minimum(u0, u1)
            q = gr * KC + u
            pos = jnp.int32(BIGI)
            for s in range(CW // 16):
                v = chunks[b][q, pl.ds(s * 16, 16)]
                pos = jnp.minimum(pos, jnp.min(
                    jnp.where(v == msp, s * 16 + iota, BIGI)))
            psp = jnp.full((16,), pos)
            nm = jnp.full((16,), NEG)
            for s in range(CW // 16):
                v = chunks[b][q, pl.ds(s * 16, 16)]
                kv = jnp.where((s * 16 + iota) == psp, NEG, v)
                chunks[b][q, pl.ds(s * 16, 16)] = kv
                nm = jnp.maximum(nm, kv)
            nmax = jnp.max(nm)
            usp = jnp.full((16,), u)
            rm0 = jnp.where(iota == usp, nmax, rm0)
            rm1 = jnp.where((iota + (KC - 16)) == usp, nmax, rm1)
            col = _sld1(cid_v, r * KC + u) * CW + pos
            sst(selv[b], gr * K + i, m)
            sst(selc[b], gr * K + i, col)
            return rm0, rm1

        rms = []
        for gr in range(G):
            r = g * G + gr
            rms.append(cmax_v[pl.ds(r * KC, 16)])
            rms.append(cmax_v[pl.ds(r * KC + KC - 16, 16)])

        def extract(i, carry):
            out = []
            for gr in range(G):
                a, c = row_step(gr, i, carry[2 * gr], carry[2 * gr + 1])
                out += [a, c]
            return tuple(out)
        lax.fori_loop(0, K, extract, tuple(rms))

    def fire_wd(b):
        for o in WCOPY:
            wcol[b][pl.ds(o, 16)] = selc[b][pl.ds(o, 16)]
        pltpu.async_copy(wd_hbm.at[wcol[b]], wrows[b], semw[b])

    def wait_wd(b):
        pltpu.make_async_copy(wd_hbm.at[wcol[b]], wrows[b], semw[b]).wait()

    def decode_group(b, g):
        def drow(gr, _):
            vals = [jnp.full((16,), _sld1(selv[b], gr * K + k))
                    for k in range(K)]
            for s in range(D_IN // 16):
                acc = bd_v[pl.ds(s * 16, 16)]
                for k in range(K):
                    acc = acc + vals[k] * wrows[b][gr * K + k,
                                                   pl.ds(s * 16, 16)]
                outbuf[b][gr, pl.ds(s * 16, 16)] = acc
            return 0
        lax.fori_loop(0, G, drow, 0)
        pltpu.async_copy(outbuf[b],
                         out_hbm.at[pl.ds(base + g * G, G)], semo[b])

    def drain_out(b, g):
        pltpu.make_async_copy(outbuf[b],
                              out_hbm.at[pl.ds(base + g * G, G)],
                              semo[b]).wait()

    for b in (0, 1):
        build_gidx(b, b)
        fire_chunks(b)

    def gg_body(gg, _):
        for b in (0, 1):
            g = gg * 2 + b
            bp = 1 - b
            wait_chunks(b)
            select_group(b, g)
            fire_wd(b)

            @pl.when(g + 2 < ngrp)
            def _prefetch():
                build_gidx(b, g + 2)
                fire_chunks(b)

            @pl.when(g >= 1)
            def _decode_prev():
                @pl.when(g >= 3)
                def _drain_prev():
                    drain_out(bp, g - 3)
                wait_wd(bp)
                decode_group(bp, g - 1)
        return 0
    lax.fori_loop(0, ngrp // 2, gg_body, 0)

    drain_out(1, ngrp - 3)
    wait_wd(1)
    decode_group(1, ngrp - 1)
    drain_out(0, ngrp - 2)
    drain_out(1, ngrp - 1)
  return _sc_body


def _decode_sc(latc, cid, cmax, Wd, b_dec, nrows):
    rpw = nrows // NWORKERS
    ngrp = rpw // G
    mesh = plsc.VectorSubcoreMesh(core_axis_name="c", subcore_axis_name="s")
    f = functools.partial(
        pl.kernel,
        out_type=jax.ShapeDtypeStruct((nrows, D_IN), jnp.float32),
        mesh=mesh,
        compiler_params=pltpu.CompilerParams(needs_layout_passes=False),
        scratch_types=[
            pltpu.VMEM((rpw * KC + 16,), jnp.int32),
            pltpu.VMEM((rpw * KC + 16,), jnp.float32),
            pltpu.VMEM((D_IN,), jnp.float32),
            pltpu.VMEM((G * KC,), jnp.int32),
            pltpu.VMEM((G * KC,), jnp.int32),
            pltpu.VMEM((G * KC, CW), jnp.float32),
            pltpu.VMEM((G * KC, CW), jnp.float32),
            pltpu.VMEM((G * K + 16,), jnp.float32),
            pltpu.VMEM((G * K + 16,), jnp.float32),
            pltpu.VMEM((G * K + 16,), jnp.int32),
            pltpu.VMEM((G * K + 16,), jnp.int32),
            pltpu.VMEM((G * K,), jnp.int32),
            pltpu.VMEM((G * K,), jnp.int32),
            pltpu.VMEM((G * K, D_IN), jnp.float32),
            pltpu.VMEM((G * K, D_IN), jnp.float32),
            pltpu.VMEM((G, D_IN), jnp.float32),
            pltpu.VMEM((G, D_IN), jnp.float32),
            pltpu.SemaphoreType.DMA,
            pltpu.SemaphoreType.DMA,
            pltpu.SemaphoreType.DMA,
            pltpu.SemaphoreType.DMA,
            pltpu.SemaphoreType.DMA,
            pltpu.SemaphoreType.DMA,
        ],
    )
    return f(_make_sc_body(rpw, ngrp))(latc, cid, cmax, Wd, b_dec)


def kernel(x, W_enc, b_enc, W_dec, b_dec):
    Wd = W_dec.T.reshape(D_LAT, D_IN)
    h = ROWS // 4
    outs = []
    for p in range(4):
        xh = x[p * h:(p + 1) * h]
        lat, cid, cmax = _encode(xh, W_enc, b_enc, h)
        latc = lat.reshape(h * NCHUNKS, CW)
        outs.append(_decode_sc(latc, cid.reshape(h * KC),
                               cmax.reshape(h * KC), Wd, b_dec, h))
    return jnp.concatenate(outs, axis=0)


# speedup vs baseline: 1.9000x; 1.0446x over previous
import functools

import jax
import jax.numpy as jnp
from jax import lax
from jax.experimental import pallas as pl
from jax.experimental.pallas import tpu as pltpu
from jax.experimental.pallas import tpu_sc as plsc

ROWS = 8192
D_IN = 768
D_LAT = 12288
K = 20
KC = 24
CW = 128
NCHUNKS = D_LAT // CW
BLK = 1024
CHUNK = 1024
NC = D_LAT // CHUNK
NEG = -3.4e38
BIGI = 2147480000

NWORKERS = 32
RPW = ROWS // NWORKERS
G = 2
NGRP = RPW // G
WCOPY = (0, 16, G * K - 16)


def _enc_body(x_ref, we_ref, be_ref, lat_ref):
    lat_ref[...] = lax.dot_general(
        x_ref[...], we_ref[...], (((1,), (1,)), ((), ())),
        preferred_element_type=jnp.float32,
    ) + be_ref[...][None, :]


BLK2 = 256


def _sel_body(lat_ref, cid_ref, cmax_ref):
    M = jnp.max(lat_ref[...].reshape(BLK2, NCHUNKS, CW), axis=2)
    iota = lax.broadcasted_iota(jnp.int32, (BLK2, NCHUNKS), 1)
    for i in range(KC):
        m = jnp.max(M, axis=1, keepdims=True)
        cid = jnp.min(jnp.where(M == m, iota, NCHUNKS),
                      axis=1, keepdims=True)
        M = jnp.where(iota == cid, NEG, M)
        cid_ref[:, pl.ds(i, 1)] = cid
        cmax_ref[:, pl.ds(i, 1)] = m


def _encode(x, W_enc, b_enc, nrows):
    lat = pl.pallas_call(
        _enc_body,
        grid=(nrows // BLK, NC),
        in_specs=[
            pl.BlockSpec((BLK, D_IN), lambda r, c: (r, 0)),
            pl.BlockSpec((CHUNK, D_IN), lambda r, c: (c, 0)),
            pl.BlockSpec((CHUNK,), lambda r, c: (c,)),
        ],
        out_specs=pl.BlockSpec((BLK, CHUNK), lambda r, c: (r, c)),
        out_shape=jax.ShapeDtypeStruct((nrows, D_LAT), jnp.float32),
    )(x, W_enc, b_enc)
    cid, cmax = pl.pallas_call(
        _sel_body,
        grid=(nrows // BLK2,),
        in_specs=[pl.BlockSpec((BLK2, D_LAT), lambda r: (r, 0))],
        out_specs=[
            pl.BlockSpec((BLK2, KC), lambda r: (r, 0)),
            pl.BlockSpec((BLK2, KC), lambda r: (r, 0)),
        ],
        out_shape=[
            jax.ShapeDtypeStruct((nrows, KC), jnp.int32),
            jax.ShapeDtypeStruct((nrows, KC), jnp.float32),
        ],
    )(lat)
    return lat, cid, cmax


def _splat(val):
    return jnp.full((16,), val)


def _sld1(ref, i):
    return ref[pl.ds(i, 16)][0]


def _make_sc_body(rpw, ngrp):
  def _sc_body(latc, cid_hbm, cmax_hbm, wd_hbm, bd_hbm, out_hbm,
             cid_v, cmax_v, bd_v, gidx0_v, gidx1_v, chunks0_v, chunks1_v,
             selv0_v, selv1_v, selc0_v, selc1_v, wcol0_v, wcol1_v,
             wrows0_v, wrows1_v, outbuf0_v, outbuf1_v,
             semc0, semc1, semw0, semw1, semo0, semo1):
    wid = lax.axis_index("s") * 2 + lax.axis_index("c")
    base = wid * rpw
    pltpu.sync_copy(cid_hbm.at[pl.ds(base * KC, rpw * KC)],
                    cid_v.at[pl.ds(0, rpw * KC)])
    pltpu.sync_copy(cmax_hbm.at[pl.ds(base * KC, rpw * KC)],
                    cmax_v.at[pl.ds(0, rpw * KC)])
    pltpu.sync_copy(bd_hbm, bd_v)
    iota = lax.broadcasted_iota(jnp.int32, (16,), 0)
    lane0 = iota == 0
    semc = (semc0, semc1)
    semw = (semw0, semw1)
    semo = (semo0, semo1)
    gidx = (gidx0_v, gidx1_v)
    chunks = (chunks0_v, chunks1_v)
    selv = (selv0_v, selv1_v)
    selc = (selc0_v, selc1_v)
    wcol = (wcol0_v, wcol1_v)
    wrows = (wrows0_v, wrows1_v)
    outbuf = (outbuf0_v, outbuf1_v)

    def sst(ref, i, val):
        cur = ref[pl.ds(i, 16)]
        ref[pl.ds(i, 16)] = jnp.where(lane0, jnp.full((16,), val), cur)

    def build_gidx(b, g):
        def row(gr, _):
            r = g * G + gr
            rowbase = _splat((base + r) * NCHUNKS)
            ca = cid_v[pl.ds(r * KC, 16)]
            cb = cid_v[pl.ds(r * KC + KC - 16, 16)]
            gidx[b][pl.ds(gr * KC, 16)] = rowbase + ca
            gidx[b][pl.ds(gr * KC + KC - 16, 16)] = rowbase + cb
            return 0
        lax.fori_loop(0, G, row, 0)

    def fire_chunks(b):
        pltpu.async_copy(latc.at[gidx[b]], chunks[b], semc[b])

    def wait_chunks(b):
        pltpu.make_async_copy(latc.at[gidx[b]], chunks[b], semc[b]).wait()

    def select_group(b, g):
        def row_step(gr, i, rm0, rm1):
            r = g * G + gr
            m = jnp.maximum(jnp.max(rm0), jnp.max(rm1))
            msp = jnp.full((16,), m)
            u0 = jnp.min(jnp.where(rm0 == msp, iota, BIGI))
            u1 = jnp.min(jnp.where(rm1 == msp, iota + (KC - 16), BIGI))
            u = jnp.minimum(u0, u1)
            q = gr * KC + u
            pos = jnp.int32(BIGI)
            for s in range(CW // 16):
                v = chunks[b][q, pl.ds(s * 16, 16)]
                pos = jnp.minimum(pos, jnp.min(
                    jnp.where(v == msp, s * 16 + iota, BIGI)))
            psp = jnp.full((16,), pos)
            nm = jnp.full((16,), NEG)
            for s in range(CW // 16):
                v = chunks[b][q, pl.ds(s * 16, 16)]
                kv = jnp.where((s * 16 + iota) == psp, NEG, v)
                chunks[b][q, pl.ds(s * 16, 16)] = kv
                nm = jnp.maximum(nm, kv)
            nmax = jnp.max(nm)
            usp = jnp.full((16,), u)
            rm0 = jnp.where(iota == usp, nmax, rm0)
            rm1 = jnp.where((iota + (KC - 16)) == usp, nmax, rm1)
            col = _sld1(cid_v, r * KC + u) * CW + pos
            sst(selv[b], gr * K + i, m)
            sst(selc[b], gr * K + i, col)
            return rm0, rm1

        rms = []
        for gr in range(G):
            r = g * G + gr
            rms.append(cmax_v[pl.ds(r * KC, 16)])
            rms.append(cmax_v[pl.ds(r * KC + KC - 16, 16)])

        def extract(i, carry):
            out = []
            for gr in range(G):
                a, c = row_step(gr, i, carry[2 * gr], carry[2 * gr + 1])
                out += [a, c]
            return tuple(out)
        lax.fori_loop(0, K, extract, tuple(rms))

    def fire_wd(b):
        for o in WCOPY:
            wcol[b][pl.ds(o, 16)] = selc[b][pl.ds(o, 16)]
        pltpu.async_copy(wd_hbm.at[wcol[b]], wrows[b], semw[b])

    def wait_wd(b):
        pltpu.make_async_copy(wd_hbm.at[wcol[b]], wrows[b], semw[b]).wait()

    def decode_group(b, g):
        def drow(gr, _):
            vals = [jnp.full((16,), _sld1(selv[b], gr * K + k))
                    for k in range(K)]
            for s in range(D_IN // 16):
                acc = bd_v[pl.ds(s * 16, 16)]
                for k in range(K):
                    acc = acc + vals[k] * wrows[b][gr * K + k,
                                                   pl.ds(s * 16, 16)]
                outbuf[b][gr, pl.ds(s * 16, 16)] = acc
            return 0
        lax.fori_loop(0, G, drow, 0)
        pltpu.async_copy(outbuf[b],
                         out_hbm.at[pl.ds(base + g * G, G)], semo[b])

    def drain_out(b, g):
        pltpu.make_async_copy(outbuf[b],
                              out_hbm.at[pl.ds(base + g * G, G)],
                              semo[b]).wait()

    for b in (0, 1):
        build_gidx(b, b)
        fire_chunks(b)

    def gg_body(gg, _):
        for b in (0, 1):
            g = gg * 2 + b
            bp = 1 - b
            wait_chunks(b)
            select_group(b, g)
            fire_wd(b)

            @pl.when(g + 2 < ngrp)
            def _prefetch():
                build_gidx(b, g + 2)
                fire_chunks(b)

            @pl.when(g >= 1)
            def _decode_prev():
                @pl.when(g >= 3)
                def _drain_prev():
                    drain_out(bp, g - 3)
                wait_wd(bp)
                decode_group(bp, g - 1)
        return 0
    lax.fori_loop(0, ngrp // 2, gg_body, 0)

    drain_out(1, ngrp - 3)
    wait_wd(1)
    decode_group(1, ngrp - 1)
    drain_out(0, ngrp - 2)
    drain_out(1, ngrp - 1)
  return _sc_body


def _decode_sc(latc, cid, cmax, Wd, b_dec, nrows):
    rpw = nrows // NWORKERS
    ngrp = rpw // G
    mesh = plsc.VectorSubcoreMesh(core_axis_name="c", subcore_axis_name="s")
    f = functools.partial(
        pl.kernel,
        out_type=jax.ShapeDtypeStruct((nrows, D_IN), jnp.float32),
        mesh=mesh,
        compiler_params=pltpu.CompilerParams(needs_layout_passes=False),
        scratch_types=[
            pltpu.VMEM((rpw * KC + 16,), jnp.int32),
            pltpu.VMEM((rpw * KC + 16,), jnp.float32),
            pltpu.VMEM((D_IN,), jnp.float32),
            pltpu.VMEM((G * KC,), jnp.int32),
            pltpu.VMEM((G * KC,), jnp.int32),
            pltpu.VMEM((G * KC, CW), jnp.float32),
            pltpu.VMEM((G * KC, CW), jnp.float32),
            pltpu.VMEM((G * K + 16,), jnp.float32),
            pltpu.VMEM((G * K + 16,), jnp.float32),
            pltpu.VMEM((G * K + 16,), jnp.int32),
            pltpu.VMEM((G * K + 16,), jnp.int32),
            pltpu.VMEM((G * K,), jnp.int32),
            pltpu.VMEM((G * K,), jnp.int32),
            pltpu.VMEM((G * K, D_IN), jnp.float32),
            pltpu.VMEM((G * K, D_IN), jnp.float32),
            pltpu.VMEM((G, D_IN), jnp.float32),
            pltpu.VMEM((G, D_IN), jnp.float32),
            pltpu.SemaphoreType.DMA,
            pltpu.SemaphoreType.DMA,
            pltpu.SemaphoreType.DMA,
            pltpu.SemaphoreType.DMA,
            pltpu.SemaphoreType.DMA,
            pltpu.SemaphoreType.DMA,
        ],
    )
    return f(_make_sc_body(rpw, ngrp))(latc, cid, cmax, Wd, b_dec)


def kernel(x, W_enc, b_enc, W_dec, b_dec):
    Wd = W_dec.T.reshape(D_LAT, D_IN)
    h = ROWS // 8
    outs = []
    for p in range(8):
        xh = x[p * h:(p + 1) * h]
        lat, cid, cmax = _encode(xh, W_enc, b_enc, h)
        latc = lat.reshape(h * NCHUNKS, CW)
        outs.append(_decode_sc(latc, cid.reshape(h * KC),
                               cmax.reshape(h * KC), Wd, b_dec, h))
    return jnp.concatenate(outs, axis=0)
